# baseline (device time: 45712 ns/iter reference)
import jax
import jax.numpy as jnp
from jax import lax
from jax.experimental import pallas as pl
from jax.experimental.pallas import tpu as pltpu

B, SQ, H, D = 4, 32, 8, 128
SKV = 4096
SCALE = D ** -0.5

DeviceIdType = getattr(pl, "DeviceIdType", None) or pltpu.DeviceIdType
semaphore_signal = getattr(pl, "semaphore_signal", None) or pltpu.semaphore_signal
semaphore_wait = getattr(pl, "semaphore_wait", None) or pltpu.semaphore_wait


def kernel(Q, K, V):
    def body(q_ref, k_ref, v_ref, out_ref,
             kbuf, vbuf,
             obuf, mbuf, lbuf,
             pobuf, pmbuf, plbuf,
             hbuf,
             dma_sems, x_send, x_recv, ag_send, ag_recv):
        my_x = lax.axis_index("x")
        my_y = lax.axis_index("y")
        my_z = lax.axis_index("z")
        g = 4 * my_y + my_z
        s0 = 2 * my_z + my_y
        x_peer = (1 - my_x, my_y, my_z)
        y_peer = (my_x, 1 - my_y, my_z)
        z1_peer = (my_x, my_y, my_z ^ 1)
        z2_peer = (my_x, my_y, my_z ^ 2)
        peers = [x_peer, y_peer, z1_peer, z2_peer]

        barrier_sem = pltpu.get_barrier_semaphore()
        for p in peers:
            semaphore_signal(barrier_sem, inc=1, device_id=p,
                             device_id_type=DeviceIdType.MESH)
        semaphore_wait(barrier_sem, len(peers))

        copies = []
        for b in range(B):
            cp_k = pltpu.make_async_copy(
                k_ref.at[b, :, pl.ds(g, 1), :], kbuf.at[b],
                dma_sems.at[2 * b])
            cp_v = pltpu.make_async_copy(
                v_ref.at[b, :, pl.ds(g, 1), :], vbuf.at[b],
                dma_sems.at[2 * b + 1])
            cp_k.start()
            cp_v.start()
            copies.append((cp_k, cp_v))

        for b in range(B):
            q = q_ref[b, :, pl.ds(g, 1), :][:, 0, :]
            q = (q * SCALE).astype(jnp.bfloat16)
            copies[b][0].wait()
            k = kbuf[b, :, 0, :].astype(jnp.bfloat16)
            s = lax.dot_general(
                q, k, (((1,), (1,)), ((), ())),
                preferred_element_type=jnp.float32,
            )
            m = jnp.max(s, axis=1, keepdims=True)
            p = jnp.exp(s - m)
            copies[b][1].wait()
            pv = lax.dot_general(
                p.astype(jnp.bfloat16),
                vbuf[b, :, 0, :].astype(jnp.bfloat16),
                (((1,), (0,)), ((), ())),
                preferred_element_type=jnp.float32,
            )
            mbuf[b] = m
            lbuf[b] = jnp.sum(p, axis=1, keepdims=True)
            obuf[b] = pv

        rdmas = []
        for i, (src, dst) in enumerate(
            [(obuf, pobuf), (mbuf, pmbuf), (lbuf, plbuf)]
        ):
            rdma = pltpu.make_async_remote_copy(
                src_ref=src, dst_ref=dst,
                send_sem=x_send.at[i], recv_sem=x_recv.at[i],
                device_id=x_peer, device_id_type=DeviceIdType.MESH,
            )
            rdma.start()
            rdmas.append(rdma)
        for rdma in rdmas:
            rdma.wait()

        m1, m2 = mbuf[...], pmbuf[...]
        l1, l2 = lbuf[...], plbuf[...]
        mm = jnp.maximum(m1, m2)
        a1 = jnp.exp(m1 - mm)
        a2 = jnp.exp(m2 - mm)
        ll = l1 * a1 + l2 * a2
        merged = (obuf[...] * a1 + pobuf[...] * a2) / ll
        hbuf[pl.ds(s0, 1)] = merged[None]

        for p, partner in enumerate([y_peer, z1_peer, z2_peer]):
            size = 1 << p
            base = s0 & (~(size - 1) & 7)
            ag = pltpu.make_async_remote_copy(
                src_ref=hbuf.at[pl.ds(base, size)],
                dst_ref=hbuf.at[pl.ds(base, size)],
                send_sem=ag_send.at[p], recv_sem=ag_recv.at[p],
                device_id=partner, device_id_type=DeviceIdType.MESH,
            )
            ag.start()
            ag.wait()

        for s_ in range(H):
            h = 4 * (s_ & 1) + (s_ >> 1)
            for b in range(B):
                out_ref[b, :, h, :] = hbuf[s_, b]

    return pl.pallas_call(
        body,
        out_shape=jax.ShapeDtypeStruct((B, SQ, H, D), jnp.float32),
        in_specs=[
            pl.BlockSpec(memory_space=pltpu.VMEM),
            pl.BlockSpec(memory_space=pl.ANY),
            pl.BlockSpec(memory_space=pl.ANY),
        ],
        out_specs=pl.BlockSpec(memory_space=pltpu.VMEM),
        scratch_shapes=[
            pltpu.VMEM((B, SKV, 1, D), jnp.float32),
            pltpu.VMEM((B, SKV, 1, D), jnp.float32),
            pltpu.VMEM((B, SQ, D), jnp.float32),
            pltpu.VMEM((B, SQ, 1), jnp.float32),
            pltpu.VMEM((B, SQ, 1), jnp.float32),
            pltpu.VMEM((B, SQ, D), jnp.float32),
            pltpu.VMEM((B, SQ, 1), jnp.float32),
            pltpu.VMEM((B, SQ, 1), jnp.float32),
            pltpu.VMEM((H, B, SQ, D), jnp.float32),
            pltpu.SemaphoreType.DMA((2 * B,)),
            pltpu.SemaphoreType.DMA((3,)),
            pltpu.SemaphoreType.DMA((3,)),
            pltpu.SemaphoreType.DMA((3,)),
            pltpu.SemaphoreType.DMA((3,)),
        ],
        compiler_params=pltpu.CompilerParams(
            collective_id=0,
            vmem_limit_bytes=100 * 1024 * 1024,
        ),
    )(Q, K, V)


# device time: 25488 ns/iter; 1.7935x vs baseline; 1.7935x over previous
import jax
import jax.numpy as jnp
from jax import lax
from jax.experimental import pallas as pl
from jax.experimental.pallas import tpu as pltpu

B, SQ, H, D = 4, 32, 8, 128
SKV = 4096
SCALE = D ** -0.5
N_BCAST = 7

DeviceIdType = getattr(pl, "DeviceIdType", None) or pltpu.DeviceIdType
semaphore_signal = getattr(pl, "semaphore_signal", None) or pltpu.semaphore_signal
semaphore_wait = getattr(pl, "semaphore_wait", None) or pltpu.semaphore_wait


def kernel(Q, K, V):
    def body(q_ref, k_ref, v_ref, out_ref,
             kbuf, vbuf,
             obuf, lbuf,
             pobuf, plbuf,
             dma_sems, x_send, x_recv, ag_send, ag_recv):
        my_x = lax.axis_index("x")
        my_y = lax.axis_index("y")
        my_z = lax.axis_index("z")
        g = 4 * my_y + my_z
        x_peer = (1 - my_x, my_y, my_z)
        group_peers = [
            (my_x, my_y ^ dy, my_z ^ dz)
            for dy in range(2) for dz in range(4) if (dy, dz) != (0, 0)
        ]

        copies = []
        for b in range(B):
            cp_k = pltpu.make_async_copy(
                k_ref.at[b, :, pl.ds(g, 1), :], kbuf.at[b],
                dma_sems.at[2 * b])
            cp_v = pltpu.make_async_copy(
                v_ref.at[b, :, pl.ds(g, 1), :], vbuf.at[b],
                dma_sems.at[2 * b + 1])
            cp_k.start()
            cp_v.start()
            copies.append((cp_k, cp_v))

        barrier_sem = pltpu.get_barrier_semaphore()
        for p in [x_peer] + group_peers:
            semaphore_signal(barrier_sem, inc=1, device_id=p,
                             device_id_type=DeviceIdType.MESH)
        semaphore_wait(barrier_sem, 1 + N_BCAST)

        x_rdmas = []
        ag_rdmas = []

        def merge_and_bcast(b):
            for rdma in x_rdmas[2 * b:2 * b + 2]:
                rdma.wait()
            merged = (obuf[b] + pobuf[b]) / (lbuf[b] + plbuf[b])
            out_ref[b, :, pl.ds(g, 1), :] = merged[:, None, :]
            for i, partner in enumerate(group_peers):
                ag = pltpu.make_async_remote_copy(
                    src_ref=out_ref.at[pl.ds(b, 1), :, pl.ds(g, 1), :],
                    dst_ref=out_ref.at[pl.ds(b, 1), :, pl.ds(g, 1), :],
                    send_sem=ag_send.at[b, i], recv_sem=ag_recv.at[b, i],
                    device_id=partner, device_id_type=DeviceIdType.MESH,
                )
                ag.start()
                ag_rdmas.append(ag)

        for b in range(B):
            q = q_ref[b, :, pl.ds(g, 1), :][:, 0, :] * SCALE
            copies[b][0].wait()
            k = kbuf[b, :, 0, :]
            s = lax.dot_general(
                q, k, (((1,), (1,)), ((), ())),
                preferred_element_type=jnp.float32,
            )
            p = jnp.exp(s)
            copies[b][1].wait()
            pv = lax.dot_general(
                p, vbuf[b, :, 0, :], (((1,), (0,)), ((), ())),
                preferred_element_type=jnp.float32,
            )
            lbuf[b] = jnp.sum(p, axis=1, keepdims=True)
            obuf[b] = pv
            for i, (src, dst) in enumerate(
                [(obuf, pobuf), (lbuf, plbuf)]
            ):
                rdma = pltpu.make_async_remote_copy(
                    src_ref=src.at[b], dst_ref=dst.at[b],
                    send_sem=x_send.at[b, i], recv_sem=x_recv.at[b, i],
                    device_id=x_peer, device_id_type=DeviceIdType.MESH,
                )
                rdma.start()
                x_rdmas.append(rdma)
            if b > 0:
                merge_and_bcast(b - 1)
        merge_and_bcast(B - 1)
        for ag in ag_rdmas:
            ag.wait()

    return pl.pallas_call(
        body,
        out_shape=jax.ShapeDtypeStruct((B, SQ, H, D), jnp.float32),
        in_specs=[
            pl.BlockSpec(memory_space=pltpu.VMEM),
            pl.BlockSpec(memory_space=pl.ANY),
            pl.BlockSpec(memory_space=pl.ANY),
        ],
        out_specs=pl.BlockSpec(memory_space=pltpu.VMEM),
        scratch_shapes=[
            pltpu.VMEM((B, SKV, 1, D), jnp.float32),
            pltpu.VMEM((B, SKV, 1, D), jnp.float32),
            pltpu.VMEM((B, SQ, D), jnp.float32),
            pltpu.VMEM((B, SQ, 1), jnp.float32),
            pltpu.VMEM((B, SQ, D), jnp.float32),
            pltpu.VMEM((B, SQ, 1), jnp.float32),
            pltpu.SemaphoreType.DMA((2 * B,)),
            pltpu.SemaphoreType.DMA((B, 2)),
            pltpu.SemaphoreType.DMA((B, 2)),
            pltpu.SemaphoreType.DMA((B, N_BCAST)),
            pltpu.SemaphoreType.DMA((B, N_BCAST)),
        ],
        compiler_params=pltpu.CompilerParams(
            collective_id=0,
            vmem_limit_bytes=100 * 1024 * 1024,
        ),
    )(Q, K, V)
